# baseline (device time: 99982 ns/iter reference)
import jax
import jax.numpy as jnp
from jax import lax
from jax.experimental import pallas as pl
from jax.experimental.pallas import tpu as pltpu

PIECES = {0: (0, 2736), 1: (2736, 2736), 2: (5472, 2720)}
PIECE_OF = {(0, 0): 0, (0, 1): 1, (1, 0): 2, (1, 1): 0}
MAXROWS = 2736


def _chunks(start, total):
    sizes = [64, 64, 128]
    rest = total - sum(sizes)
    sizes += [256] * (rest // 256)
    if rest % 256:
        sizes.append(rest % 256)
    assert sum(sizes) == total and all(s % 16 == 0 for s in sizes)
    out = []
    r = start
    for s in sizes:
        out.append((r, s))
        r += s
    return out


MAXCH = max(len(_chunks(*PIECES[p])) for p in PIECES)


def kernel(x):
    m, n = x.shape

    def body(
        x_ref,
        out_ref,
        xv_ref,
        zsend_ref,
        comm_ref,
        ldx,
        sz, rz, sx, rx, sy, ry,
    ):
        my_x = lax.axis_index("x")
        my_y = lax.axis_index("y")
        my_z = lax.axis_index("z")

        barrier_sem = pltpu.get_barrier_semaphore()
        for nbr in (
            (my_x, my_y, 1 - my_z),
            (1 - my_x, my_y, my_z),
            (my_x, 1 - my_y, my_z),
        ):
            pl.semaphore_signal(
                barrier_sem,
                inc=1,
                device_id=nbr,
                device_id_type=pl.DeviceIdType.MESH,
            )
        pl.semaphore_wait(barrier_sem, 3)

        def emit_column(cx, cy):
            own_p = PIECE_OF[(cx, cy)]
            xin_p = PIECE_OF[(1 - cx, cy)]
            diag = cx == cy
            ynbr_diag = cx == (1 - cy)
            yin_p = (
                PIECE_OF[(1 - cx, 1 - cy)]
                if ynbr_diag
                else PIECE_OF[(cx, 1 - cy)]
            )
            own_ch = _chunks(*PIECES[own_p])
            xin_ch = _chunks(*PIECES[xin_p])
            yin_ch = _chunks(*PIECES[yin_p])
            o0 = PIECES[own_p][0]
            n_own, n_xin, n_yin = len(own_ch), len(xin_ch), len(yin_ch)

            z_dev = (cx, cy, 1 - my_z)
            x_dev = (1 - cx, cy, my_z)
            y_dev = (cx, 1 - cy, my_z)

            ld = []
            for j, (g, nr) in enumerate(own_ch):
                c = pltpu.make_async_copy(
                    x_ref.at[pl.ds(g, nr)],
                    xv_ref.at[pl.ds(g - o0, nr)],
                    ldx.at[j],
                )
                c.start()
                ld.append(c)

            z_rdmas = []
            for j, (g, nr) in enumerate(own_ch):
                ld[j].wait()
                zsend_ref[pl.ds(g - o0, nr), :] = xv_ref[
                    pl.ds(g - o0, nr), :
                ].astype(jnp.bfloat16)
                r = pltpu.make_async_remote_copy(
                    src_ref=zsend_ref.at[pl.ds(g - o0, nr)],
                    dst_ref=comm_ref.at[pl.ds(g - o0, nr)],
                    send_sem=sz.at[j],
                    recv_sem=rz.at[j],
                    device_id=z_dev,
                    device_id_type=pl.DeviceIdType.MESH,
                )
                r.start()
                z_rdmas.append(r)

            def fwd(g, nr, ssem, rsem, dev):
                return pltpu.make_async_remote_copy(
                    src_ref=out_ref.at[pl.ds(g, nr)],
                    dst_ref=out_ref.at[pl.ds(g, nr)],
                    send_sem=ssem,
                    recv_sem=rsem,
                    device_id=dev,
                    device_id_type=pl.DeviceIdType.MESH,
                )

            x_recvs = [
                fwd(g, nr, sx.at[j], rx.at[j], x_dev)
                for j, (g, nr) in enumerate(xin_ch)
            ]
            y_recvs = [
                fwd(g, nr, sy.at[j], ry.at[j], y_dev)
                for j, (g, nr) in enumerate(yin_ch)
            ]

            sends = []
            for t in range(max(n_own, n_xin if diag else 0)):
                if t < n_own:
                    g, nr = own_ch[t]
                    loc = g - o0
                    z_rdmas[t].wait()
                    out_ref[pl.ds(g, nr), :] = (
                        zsend_ref[pl.ds(loc, nr), :]
                        + comm_ref[pl.ds(loc, nr), :]
                    )
                    xs = fwd(g, nr, sx.at[t], rx.at[t], x_dev)
                    xs.start()
                    sends.append(xs)
                    if not diag:
                        ys = fwd(g, nr, sy.at[t], ry.at[t], y_dev)
                        ys.start()
                        sends.append(ys)
                if diag and t < n_xin:
                    g, nr = xin_ch[t]
                    x_recvs[t].wait_recv()
                    ys = fwd(g, nr, sy.at[t], ry.at[t], y_dev)
                    ys.start()
                    sends.append(ys)

            if not diag:
                for r in x_recvs:
                    r.wait_recv()
            for r in y_recvs:
                r.wait_recv()
            for s in sends:
                s.wait_send()

        for cx in (0, 1):
            for cy in (0, 1):

                @pl.when((my_x == cx) & (my_y == cy))
                def _(cx=cx, cy=cy):
                    emit_column(cx, cy)

    return pl.pallas_call(
        body,
        out_shape=jax.ShapeDtypeStruct((m, n), jnp.bfloat16),
        in_specs=[pl.BlockSpec(memory_space=pl.ANY)],
        out_specs=pl.BlockSpec(memory_space=pltpu.VMEM),
        scratch_shapes=[
            pltpu.VMEM((MAXROWS, n), jnp.float32),
            pltpu.VMEM((MAXROWS, n), jnp.bfloat16),
            pltpu.VMEM((MAXROWS, n), jnp.bfloat16),
            pltpu.SemaphoreType.DMA((MAXCH,)),
            pltpu.SemaphoreType.DMA((MAXCH,)),
            pltpu.SemaphoreType.DMA((MAXCH,)),
            pltpu.SemaphoreType.DMA((MAXCH,)),
            pltpu.SemaphoreType.DMA((MAXCH,)),
            pltpu.SemaphoreType.DMA((MAXCH,)),
            pltpu.SemaphoreType.DMA((MAXCH,)),
        ],
        compiler_params=pltpu.CompilerParams(
            collective_id=0, vmem_limit_bytes=100 * 1024 * 1024
        ),
    )(x)


# device time: 95119 ns/iter; 1.0511x vs baseline; 1.0511x over previous
import jax
import jax.numpy as jnp
from jax import lax
from jax.experimental import pallas as pl
from jax.experimental.pallas import tpu as pltpu

PIECES = {0: (0, 2736), 1: (2736, 2736), 2: (5472, 2720)}
PIECE_OF = {(0, 0): 0, (0, 1): 1, (1, 0): 2, (1, 1): 0}
MAXROWS = 2736


def _chunks(start, total):
    sizes = [64, 64, 128]
    rest = total - sum(sizes)
    sizes += [256] * (rest // 256)
    if rest % 256:
        sizes.append(rest % 256)
    assert sum(sizes) == total and all(s % 16 == 0 for s in sizes)
    out = []
    r = start
    for s in sizes:
        out.append((r, s))
        r += s
    return out


MAXCH = max(len(_chunks(*PIECES[p])) for p in PIECES)


def kernel(x):
    m, n = x.shape

    def body(
        x_ref,
        out_ref,
        xv_ref,
        zsend_ref,
        comm_ref,
        sum_ref,
        ldx, lout,
        sz, rz, sx, rx, sy, ry,
    ):
        my_x = lax.axis_index("x")
        my_y = lax.axis_index("y")
        my_z = lax.axis_index("z")

        barrier_sem = pltpu.get_barrier_semaphore()
        for nbr in (
            (my_x, my_y, 1 - my_z),
            (1 - my_x, my_y, my_z),
            (my_x, 1 - my_y, my_z),
        ):
            pl.semaphore_signal(
                barrier_sem,
                inc=1,
                device_id=nbr,
                device_id_type=pl.DeviceIdType.MESH,
            )
        pl.semaphore_wait(barrier_sem, 3)

        def emit_column(cx, cy):
            own_p = PIECE_OF[(cx, cy)]
            xin_p = PIECE_OF[(1 - cx, cy)]
            diag = cx == cy
            ynbr_diag = cx == (1 - cy)
            yin_p = (
                PIECE_OF[(1 - cx, 1 - cy)]
                if ynbr_diag
                else PIECE_OF[(cx, 1 - cy)]
            )
            own_ch = _chunks(*PIECES[own_p])
            xin_ch = _chunks(*PIECES[xin_p])
            yin_ch = _chunks(*PIECES[yin_p])
            o0 = PIECES[own_p][0]
            n_own, n_xin, n_yin = len(own_ch), len(xin_ch), len(yin_ch)

            z_dev = (cx, cy, 1 - my_z)
            x_dev = (1 - cx, cy, my_z)
            y_dev = (cx, 1 - cy, my_z)

            ld = []
            for j, (g, nr) in enumerate(own_ch):
                c = pltpu.make_async_copy(
                    x_ref.at[pl.ds(g, nr)],
                    xv_ref.at[pl.ds(g - o0, nr)],
                    ldx.at[j],
                )
                c.start()
                ld.append(c)

            z_rdmas = []
            for j, (g, nr) in enumerate(own_ch):
                ld[j].wait()
                zsend_ref[pl.ds(g - o0, nr), :] = xv_ref[
                    pl.ds(g - o0, nr), :
                ].astype(jnp.bfloat16)
                r = pltpu.make_async_remote_copy(
                    src_ref=zsend_ref.at[pl.ds(g - o0, nr)],
                    dst_ref=comm_ref.at[pl.ds(g - o0, nr)],
                    send_sem=sz.at[j],
                    recv_sem=rz.at[j],
                    device_id=z_dev,
                    device_id_type=pl.DeviceIdType.MESH,
                )
                r.start()
                z_rdmas.append(r)

            def fwd(src_ref, g, nr, loc, ssem, rsem, dev):
                return pltpu.make_async_remote_copy(
                    src_ref=src_ref.at[pl.ds(loc, nr)],
                    dst_ref=out_ref.at[pl.ds(g, nr)],
                    send_sem=ssem,
                    recv_sem=rsem,
                    device_id=dev,
                    device_id_type=pl.DeviceIdType.MESH,
                )

            x_recvs = [
                fwd(out_ref, g, nr, g, sx.at[j], rx.at[j], x_dev)
                for j, (g, nr) in enumerate(xin_ch)
            ]
            y_recvs = [
                fwd(out_ref, g, nr, g, sy.at[j], ry.at[j], y_dev)
                for j, (g, nr) in enumerate(yin_ch)
            ]

            sends = []
            louts = []
            for t in range(max(n_own, n_xin if diag else 0)):
                if t < n_own:
                    g, nr = own_ch[t]
                    loc = g - o0
                    z_rdmas[t].wait()
                    sum_ref[pl.ds(loc, nr), :] = (
                        zsend_ref[pl.ds(loc, nr), :]
                        + comm_ref[pl.ds(loc, nr), :]
                    )
                    lc = pltpu.make_async_copy(
                        sum_ref.at[pl.ds(loc, nr)],
                        out_ref.at[pl.ds(g, nr)],
                        lout.at[t],
                    )
                    lc.start()
                    louts.append(lc)
                    xs = fwd(sum_ref, g, nr, loc, sx.at[t], rx.at[t], x_dev)
                    xs.start()
                    sends.append(xs)
                    if not diag:
                        ys = fwd(
                            sum_ref, g, nr, loc, sy.at[t], ry.at[t], y_dev
                        )
                        ys.start()
                        sends.append(ys)
                if diag and t < n_xin:
                    g, nr = xin_ch[t]
                    x_recvs[t].wait_recv()
                    ys = fwd(out_ref, g, nr, g, sy.at[t], ry.at[t], y_dev)
                    ys.start()
                    sends.append(ys)

            if not diag:
                for r in x_recvs:
                    r.wait_recv()
            for r in y_recvs:
                r.wait_recv()
            for c in louts:
                c.wait()
            for s in sends:
                s.wait_send()

        for cx in (0, 1):
            for cy in (0, 1):

                @pl.when((my_x == cx) & (my_y == cy))
                def _(cx=cx, cy=cy):
                    emit_column(cx, cy)

    return pl.pallas_call(
        body,
        out_shape=jax.ShapeDtypeStruct((m, n), jnp.bfloat16),
        in_specs=[pl.BlockSpec(memory_space=pl.ANY)],
        out_specs=pl.BlockSpec(memory_space=pl.ANY),
        scratch_shapes=[
            pltpu.VMEM((MAXROWS, n), jnp.float32),
            pltpu.VMEM((MAXROWS, n), jnp.bfloat16),
            pltpu.VMEM((MAXROWS, n), jnp.bfloat16),
            pltpu.VMEM((MAXROWS, n), jnp.bfloat16),
            pltpu.SemaphoreType.DMA((MAXCH,)),
            pltpu.SemaphoreType.DMA((MAXCH,)),
            pltpu.SemaphoreType.DMA((MAXCH,)),
            pltpu.SemaphoreType.DMA((MAXCH,)),
            pltpu.SemaphoreType.DMA((MAXCH,)),
            pltpu.SemaphoreType.DMA((MAXCH,)),
            pltpu.SemaphoreType.DMA((MAXCH,)),
            pltpu.SemaphoreType.DMA((MAXCH,)),
        ],
        compiler_params=pltpu.CompilerParams(
            collective_id=0, vmem_limit_bytes=100 * 1024 * 1024
        ),
    )(x)


# device time: 90462 ns/iter; 1.1052x vs baseline; 1.0515x over previous
import jax
import jax.numpy as jnp
from jax import lax
from jax.experimental import pallas as pl
from jax.experimental.pallas import tpu as pltpu

PIECES = {0: (0, 2736), 1: (2736, 2736), 2: (5472, 2720)}
PIECE_OF = {(0, 0): 0, (0, 1): 1, (1, 0): 2, (1, 1): 0}
MAXROWS = 2736


def _chunks(start, total):
    sizes = [64, 64, 128]
    rest = total - sum(sizes)
    sizes += [128] * (rest // 128)
    if rest % 128:
        sizes.append(rest % 128)
    assert sum(sizes) == total and all(s % 16 == 0 for s in sizes)
    out = []
    r = start
    for s in sizes:
        out.append((r, s))
        r += s
    return out


MAXCH = max(len(_chunks(*PIECES[p])) for p in PIECES)


def kernel(x):
    m, n = x.shape

    def body(
        x_ref,
        out_ref,
        xv_ref,
        zsend_ref,
        comm_ref,
        sum_ref,
        ldx, lout,
        sz, rz, sx, rx, sy, ry,
    ):
        my_x = lax.axis_index("x")
        my_y = lax.axis_index("y")
        my_z = lax.axis_index("z")

        barrier_sem = pltpu.get_barrier_semaphore()
        for nbr in (
            (my_x, my_y, 1 - my_z),
            (1 - my_x, my_y, my_z),
            (my_x, 1 - my_y, my_z),
        ):
            pl.semaphore_signal(
                barrier_sem,
                inc=1,
                device_id=nbr,
                device_id_type=pl.DeviceIdType.MESH,
            )
        pl.semaphore_wait(barrier_sem, 3)

        def emit_column(cx, cy):
            own_p = PIECE_OF[(cx, cy)]
            xin_p = PIECE_OF[(1 - cx, cy)]
            diag = cx == cy
            ynbr_diag = cx == (1 - cy)
            yin_p = (
                PIECE_OF[(1 - cx, 1 - cy)]
                if ynbr_diag
                else PIECE_OF[(cx, 1 - cy)]
            )
            own_ch = _chunks(*PIECES[own_p])
            xin_ch = _chunks(*PIECES[xin_p])
            yin_ch = _chunks(*PIECES[yin_p])
            o0 = PIECES[own_p][0]
            n_own, n_xin, n_yin = len(own_ch), len(xin_ch), len(yin_ch)

            z_dev = (cx, cy, 1 - my_z)
            x_dev = (1 - cx, cy, my_z)
            y_dev = (cx, 1 - cy, my_z)

            ld = []
            for j, (g, nr) in enumerate(own_ch):
                c = pltpu.make_async_copy(
                    x_ref.at[pl.ds(g, nr)],
                    xv_ref.at[pl.ds(g - o0, nr)],
                    ldx.at[j],
                )
                c.start()
                ld.append(c)

            z_rdmas = []
            for j, (g, nr) in enumerate(own_ch):
                ld[j].wait()
                zsend_ref[pl.ds(g - o0, nr), :] = xv_ref[
                    pl.ds(g - o0, nr), :
                ].astype(jnp.bfloat16)
                r = pltpu.make_async_remote_copy(
                    src_ref=zsend_ref.at[pl.ds(g - o0, nr)],
                    dst_ref=comm_ref.at[pl.ds(g - o0, nr)],
                    send_sem=sz.at[j],
                    recv_sem=rz.at[j],
                    device_id=z_dev,
                    device_id_type=pl.DeviceIdType.MESH,
                )
                r.start()
                z_rdmas.append(r)

            def fwd(src_ref, g, nr, loc, ssem, rsem, dev):
                return pltpu.make_async_remote_copy(
                    src_ref=src_ref.at[pl.ds(loc, nr)],
                    dst_ref=out_ref.at[pl.ds(g, nr)],
                    send_sem=ssem,
                    recv_sem=rsem,
                    device_id=dev,
                    device_id_type=pl.DeviceIdType.MESH,
                )

            x_recvs = [
                fwd(out_ref, g, nr, g, sx.at[j], rx.at[j], x_dev)
                for j, (g, nr) in enumerate(xin_ch)
            ]
            y_recvs = [
                fwd(out_ref, g, nr, g, sy.at[j], ry.at[j], y_dev)
                for j, (g, nr) in enumerate(yin_ch)
            ]

            sends = []
            louts = []
            for t in range(max(n_own, n_xin if diag else 0)):
                if t < n_own:
                    g, nr = own_ch[t]
                    loc = g - o0
                    z_rdmas[t].wait()
                    sum_ref[pl.ds(loc, nr), :] = (
                        zsend_ref[pl.ds(loc, nr), :]
                        + comm_ref[pl.ds(loc, nr), :]
                    )
                    lc = pltpu.make_async_copy(
                        sum_ref.at[pl.ds(loc, nr)],
                        out_ref.at[pl.ds(g, nr)],
                        lout.at[t],
                    )
                    lc.start()
                    louts.append(lc)
                    xs = fwd(sum_ref, g, nr, loc, sx.at[t], rx.at[t], x_dev)
                    xs.start()
                    sends.append(xs)
                    if not diag:
                        ys = fwd(
                            sum_ref, g, nr, loc, sy.at[t], ry.at[t], y_dev
                        )
                        ys.start()
                        sends.append(ys)
                if diag and t < n_xin:
                    g, nr = xin_ch[t]
                    x_recvs[t].wait_recv()
                    ys = fwd(out_ref, g, nr, g, sy.at[t], ry.at[t], y_dev)
                    ys.start()
                    sends.append(ys)

            if not diag:
                for r in x_recvs:
                    r.wait_recv()
            for r in y_recvs:
                r.wait_recv()
            for c in louts:
                c.wait()
            for s in sends:
                s.wait_send()

        for cx in (0, 1):
            for cy in (0, 1):

                @pl.when((my_x == cx) & (my_y == cy))
                def _(cx=cx, cy=cy):
                    emit_column(cx, cy)

    return pl.pallas_call(
        body,
        out_shape=jax.ShapeDtypeStruct((m, n), jnp.bfloat16),
        in_specs=[pl.BlockSpec(memory_space=pl.ANY)],
        out_specs=pl.BlockSpec(memory_space=pl.ANY),
        scratch_shapes=[
            pltpu.VMEM((MAXROWS, n), jnp.float32),
            pltpu.VMEM((MAXROWS, n), jnp.bfloat16),
            pltpu.VMEM((MAXROWS, n), jnp.bfloat16),
            pltpu.VMEM((MAXROWS, n), jnp.bfloat16),
            pltpu.SemaphoreType.DMA((MAXCH,)),
            pltpu.SemaphoreType.DMA((MAXCH,)),
            pltpu.SemaphoreType.DMA((MAXCH,)),
            pltpu.SemaphoreType.DMA((MAXCH,)),
            pltpu.SemaphoreType.DMA((MAXCH,)),
            pltpu.SemaphoreType.DMA((MAXCH,)),
            pltpu.SemaphoreType.DMA((MAXCH,)),
            pltpu.SemaphoreType.DMA((MAXCH,)),
        ],
        compiler_params=pltpu.CompilerParams(
            collective_id=0, vmem_limit_bytes=100 * 1024 * 1024
        ),
    )(x)
